# baseline (speedup 1.0000x reference)
import jax
import jax.numpy as jnp
from jax.experimental import pallas as pl
from jax.experimental.pallas import tpu as pltpu

_CP = dict(compiler_params=pltpu.CompilerParams(
    dimension_semantics=("parallel",)))

NV = 51
NP = 56
NR = NP * NP
F32 = jnp.float32
BF16 = jnp.bfloat16


def _leaky(h):
    return jnp.maximum(h, 0.01 * h)


def _row_mask(dtype=F32):
    r = jax.lax.broadcasted_iota(jnp.int32, (NR, 1), 0)
    n = r // NP
    m = r - n * NP
    return ((n < NV) & (m < NV)).astype(dtype)


def _norm_act_bf16(h, sc_ref, sh_ref):
    t = h * sc_ref[0].astype(BF16) + sh_ref[0].astype(BF16)
    return jnp.maximum(t, t * BF16(0.01))


def _gram(a):
    aug = jnp.concatenate([a, jnp.ones((NR, 8), BF16)], axis=1)
    return jax.lax.dot_general(aug, aug, (((0,), (0,)), ((), ())),
                               preferred_element_type=F32)


def _p1_kernel(x_ref, w_ref, h_ref, c_ref, q_ref):
    x = x_ref[0].astype(BF16)
    a = jnp.abs(x[:, None, :] - x[None, :, :]).reshape(NR, x.shape[1])
    a = a * _row_mask(BF16)
    h = jnp.dot(a, w_ref[...], preferred_element_type=F32)
    h_ref[0] = h.astype(BF16)
    c_ref[0] = _gram(a)
    q_ref[0, 0, :] = jnp.zeros((8,), F32)


def _mid_kernel(h_ref, sc_ref, sh_ref, w_ref, ho_ref, c_ref, q_ref):
    a = _norm_act_bf16(h_ref[0], sc_ref, sh_ref) * _row_mask(BF16)
    h = jnp.dot(a, w_ref[...], preferred_element_type=F32)
    ho_ref[0] = h.astype(BF16)
    c_ref[0] = _gram(a)
    q_ref[0, 0, :] = jnp.zeros((8,), F32)


def _p5a_kernel(h_ref, sc_ref, sh_ref, w_ref, b_ref, ho_ref):
    a = _norm_act_bf16(h_ref[0], sc_ref, sh_ref)
    ho_ref[0] = jnp.dot(a, w_ref[...],
                        preferred_element_type=F32) + b_ref[0]


def _p5b_kernel(x_ref, e_ref, wg1_ref, wg2_ref, bg_ref,
                gout_ref, gs_ref, gq_ref):
    x = x_ref[0]
    ii = jax.lax.broadcasted_iota(jnp.int32, (NP, NP), 0)
    jj = jax.lax.broadcasted_iota(jnp.int32, (NP, NP), 1)
    le = e_ref[0] + jnp.where(ii == jj, -1e8, 0.0) \
        + jnp.where(jj >= NV, -1e9, 0.0)
    mx = jnp.max(le, axis=1, keepdims=True)
    ex = jnp.exp(le - mx)
    w = ex / jnp.sum(ex, axis=1, keepdims=True)
    y = jnp.dot(w, x, preferred_element_type=F32)
    gout = (jnp.dot(x, wg1_ref[...], preferred_element_type=F32)
            + jnp.dot(y, wg2_ref[...], preferred_element_type=F32)
            + bg_ref[0])
    ni = jax.lax.broadcasted_iota(jnp.int32, (NP, 1), 0)
    gout = gout * (ni < NV).astype(F32)
    gout_ref[0] = gout
    gs_ref[0, 0, :] = jnp.sum(gout, axis=0)
    gq_ref[0, 0, :] = jnp.sum(gout * gout, axis=0)


def _p5last_kernel(x_ref, h_ref, sc_ref, sh_ref, w5_ref, b5_ref,
                   wg1_ref, wg2_ref, bg_ref, logit_ref, sig_ref):
    x = x_ref[0]
    a = _norm_act_bf16(h_ref[0], sc_ref, sh_ref)
    e = jnp.dot(a, w5_ref[...],
                preferred_element_type=F32) + b5_ref[0]
    ri = jax.lax.broadcasted_iota(jnp.int32, (NP, 1), 0)
    le = e + jnp.where(ri == 0, -1e8, 0.0) + jnp.where(ri >= NV, -1e9, 0.0)
    mx = jnp.max(le, axis=0, keepdims=True)
    ex = jnp.exp(le - mx)
    w = ex / jnp.sum(ex, axis=0, keepdims=True)
    y = jax.lax.dot_general(w, x, (((0,), (0,)), ((), ())),
                            preferred_element_type=F32)
    gl = (jnp.dot(x[0:1, :], wg1_ref[...], preferred_element_type=F32)
          + jnp.dot(y, wg2_ref[...], preferred_element_type=F32)
          + bg_ref[0])
    logit_ref[0, 0, :] = gl[0]
    sig_ref[0, 0, :] = (1.0 / (1.0 + jnp.exp(-gl)))[0]


def _bn_act_kernel(g_ref, sc_ref, sh_ref, out_ref):
    ni = jax.lax.broadcasted_iota(jnp.int32, (NP, 1), 0)
    vmask = (ni < NV).astype(F32)
    out_ref[0] = _leaky(g_ref[0] * sc_ref[0] + sh_ref[0]) * vmask


def _bspec(shape):
    nd = len(shape)
    return pl.BlockSpec(shape, lambda b: (0,) * nd)


def _gram_stats(cb, w):
    c = jnp.sum(cb, axis=0)
    f = w.shape[0]
    w32 = w.astype(F32)
    sa = c[f, :f]
    s = sa @ w32
    q = jnp.sum(w32 * (c[:f, :f] @ w32), axis=0)
    return s, q


def _finalize(s, q, b, g, beta, cnt):
    mean = (s + cnt * b) / cnt
    ex2 = (q + 2.0 * b * s + cnt * b * b) / cnt
    var = ex2 - mean * mean
    sc = g * jax.lax.rsqrt(var + 1e-5)
    sh = (b - mean) * sc + beta
    return sc.reshape(1, -1), sh.reshape(1, -1)


def _wcompute_gconv(xp, wc, gcp, last):
    B, _, F = xp.shape
    ws = [w.astype(BF16) for w in wc["w"]]
    bs = wc["b"]
    cnt = float(B * NV * NV)
    dims = [w.shape[1] for w in wc["w"]]

    def rspec(shape):
        return pl.BlockSpec((1,) + tuple(shape[1:]),
                            lambda b: (b,) + (0,) * (len(shape) - 1))

    h_shape = (B, NR, dims[0])
    c_shape = (B, F + 8, F + 8)
    d_shape = (B, 1, 8)
    h1, cb, _ = pl.pallas_call(
        _p1_kernel, grid=(B,), **_CP,
        in_specs=[rspec(xp.shape), _bspec(ws[0].shape)],
        out_specs=[rspec(h_shape), rspec(c_shape), rspec(d_shape)],
        out_shape=[jax.ShapeDtypeStruct(h_shape, BF16),
                   jax.ShapeDtypeStruct(c_shape, F32),
                   jax.ShapeDtypeStruct(d_shape, F32)])(xp, ws[0])
    s, q = _gram_stats(cb, ws[0])
    sc, sh = _finalize(s, q, bs[0], wc["g"][0], wc["beta"][0], cnt)

    h_prev = h1
    for k in range(1, 4):
        ho_shape = (B, NR, dims[k])
        co_shape = (B, dims[k - 1] + 8, dims[k - 1] + 8)
        h_next, cb, _ = pl.pallas_call(
            _mid_kernel, grid=(B,), **_CP,
            in_specs=[rspec(h_prev.shape), _bspec(sc.shape),
                      _bspec(sh.shape), _bspec(ws[k].shape)],
            out_specs=[rspec(ho_shape), rspec(co_shape), rspec(d_shape)],
            out_shape=[jax.ShapeDtypeStruct(ho_shape, BF16),
                       jax.ShapeDtypeStruct(co_shape, F32),
                       jax.ShapeDtypeStruct(d_shape, F32)])(
                h_prev, sc, sh, ws[k])
        s, q = _gram_stats(cb, ws[k])
        sc, sh = _finalize(s, q, bs[k], wc["g"][k], wc["beta"][k], cnt)
        h_prev = h_next

    b5 = bs[4].reshape(1, -1)
    wg1, wg2 = gcp["w"][:F], gcp["w"][F:]
    bg = gcp["b"].reshape(1, -1)
    Fo = wg1.shape[1]

    if last:
        o_shape = (B, 1, Fo)
        logits, sig = pl.pallas_call(
            _p5last_kernel, grid=(B,), **_CP,
            in_specs=[rspec(xp.shape),
                      pl.BlockSpec((1, NP, dims[3]), lambda b: (b, 0, 0)),
                      _bspec(sc.shape), _bspec(sh.shape),
                      _bspec(ws[4].shape), _bspec(b5.shape),
                      _bspec(wg1.shape), _bspec(wg2.shape), _bspec(bg.shape)],
            out_specs=[rspec(o_shape), rspec(o_shape)],
            out_shape=[jax.ShapeDtypeStruct(o_shape, F32)] * 2)(
                xp, h_prev, sc, sh, ws[4], b5, wg1, wg2, bg)
        return logits[:, 0, :], sig[:, 0, :]

    e_shape = (B, NR, 1)
    e_col = pl.pallas_call(
        _p5a_kernel, grid=(B,), **_CP,
        in_specs=[rspec(h_prev.shape), _bspec(sc.shape), _bspec(sh.shape),
                  _bspec(ws[4].shape), _bspec(b5.shape)],
        out_specs=rspec(e_shape),
        out_shape=jax.ShapeDtypeStruct(e_shape, F32))(
            h_prev, sc, sh, ws[4], b5)
    e_grid = e_col.reshape(B, NP, NP)

    gout_shape = (B, NP, Fo)
    gs_shape = (B, 1, Fo)
    gout, gs, gq = pl.pallas_call(
        _p5b_kernel, grid=(B,), **_CP,
        in_specs=[rspec(xp.shape), rspec(e_grid.shape),
                  _bspec(wg1.shape), _bspec(wg2.shape), _bspec(bg.shape)],
        out_specs=[rspec(gout_shape), rspec(gs_shape), rspec(gs_shape)],
        out_shape=[jax.ShapeDtypeStruct(gout_shape, F32),
                   jax.ShapeDtypeStruct(gs_shape, F32),
                   jax.ShapeDtypeStruct(gs_shape, F32)])(
            xp, e_grid, wg1, wg2, bg)
    gsc, gsh = _finalize(jnp.sum(gs[:, 0, :], 0), jnp.sum(gq[:, 0, :], 0),
                         jnp.zeros((Fo,), F32), gcp["g"], gcp["beta"],
                         float(B * NV))
    act = pl.pallas_call(
        _bn_act_kernel, grid=(B,), **_CP,
        in_specs=[rspec(gout_shape), _bspec(gsc.shape), _bspec(gsh.shape)],
        out_specs=rspec(gout_shape),
        out_shape=jax.ShapeDtypeStruct(gout_shape, F32))(gout, gsc, gsh)
    return act


def kernel(z, zi_s, labels_yi, params):
    B = z.shape[0]
    zero_pad = jnp.zeros((1, B, labels_yi.shape[2]), dtype=labels_yi.dtype)
    lab_all = jnp.concatenate([zero_pad, labels_yi], axis=0)
    z_all = jnp.concatenate([z[None], zi_s], axis=0)
    nodes = jnp.transpose(jnp.concatenate([z_all, lab_all], axis=2), (1, 0, 2))
    xp = jnp.pad(nodes, ((0, 0), (0, NP - NV), (0, 0)))
    for i in range(2):
        act = _wcompute_gconv(xp, params["wc"][i], params["gc"][i], last=False)
        xp = jnp.concatenate([xp, act], axis=2)
    logits, sig = _wcompute_gconv(xp, params["wc"][2], params["gc"][2],
                                  last=True)
    return (sig, logits)


# speedup vs baseline: 1.1601x; 1.1601x over previous
import jax
import jax.numpy as jnp
from jax.experimental import pallas as pl
from jax.experimental.pallas import tpu as pltpu

_CP = dict(compiler_params=pltpu.CompilerParams(
    dimension_semantics=("parallel",)))

NV = 51
NP = 56
NR = NP * NP
F32 = jnp.float32
BF16 = jnp.bfloat16


def _leaky(h):
    return jnp.maximum(h, 0.01 * h)


def _row_mask(dtype=F32):
    r = jax.lax.broadcasted_iota(jnp.int32, (NR, 1), 0)
    n = r // NP
    m = r - n * NP
    return ((n < NV) & (m < NV)).astype(dtype)


def _norm_act_bf16(h, sc_ref, sh_ref):
    t = h * sc_ref[0].astype(BF16) + sh_ref[0].astype(BF16)
    return jnp.maximum(t, t * BF16(0.01))


def _p1_kernel(x_ref, w_ref, h_ref, s_ref, q_ref):
    x = x_ref[0].astype(BF16)
    a = jnp.abs(x[:, None, :] - x[None, :, :]).reshape(NR, x.shape[1])
    a = a * _row_mask(BF16)
    h = jnp.dot(a, w_ref[...], preferred_element_type=F32)
    h_ref[0] = h.astype(BF16)
    s_ref[0, 0, :] = jnp.sum(h, axis=0)
    q_ref[0, 0, :] = jnp.sum(h * h, axis=0)


def _mid_kernel(h_ref, sc_ref, sh_ref, w_ref, ho_ref, s_ref, q_ref):
    a = _norm_act_bf16(h_ref[0], sc_ref, sh_ref)
    h = jnp.dot(a, w_ref[...], preferred_element_type=F32)
    ho_ref[0] = h.astype(BF16)
    s_ref[0, 0, :] = jnp.sum(h, axis=0)
    q_ref[0, 0, :] = jnp.sum(h * h, axis=0)


def _p5a_kernel(h_ref, sc_ref, sh_ref, w_ref, b_ref, ho_ref):
    a = _norm_act_bf16(h_ref[0], sc_ref, sh_ref)
    ho_ref[0] = jnp.dot(a, w_ref[...],
                        preferred_element_type=F32) + b_ref[0]


def _p5b_kernel(x_ref, e_ref, wg1_ref, wg2_ref, bg_ref,
                gout_ref, gs_ref, gq_ref):
    x = x_ref[0]
    ii = jax.lax.broadcasted_iota(jnp.int32, (NP, NP), 0)
    jj = jax.lax.broadcasted_iota(jnp.int32, (NP, NP), 1)
    le = e_ref[0] + jnp.where(ii == jj, -1e8, 0.0) \
        + jnp.where(jj >= NV, -1e9, 0.0)
    mx = jnp.max(le, axis=1, keepdims=True)
    ex = jnp.exp(le - mx)
    w = ex / jnp.sum(ex, axis=1, keepdims=True)
    y = jnp.dot(w, x, preferred_element_type=F32)
    gout = (jnp.dot(x, wg1_ref[...], preferred_element_type=F32)
            + jnp.dot(y, wg2_ref[...], preferred_element_type=F32)
            + bg_ref[0])
    ni = jax.lax.broadcasted_iota(jnp.int32, (NP, 1), 0)
    gout = gout * (ni < NV).astype(F32)
    gout_ref[0] = gout
    gs_ref[0, 0, :] = jnp.sum(gout, axis=0)
    gq_ref[0, 0, :] = jnp.sum(gout * gout, axis=0)


def _p5last_kernel(x_ref, h_ref, sc_ref, sh_ref, w5_ref, b5_ref,
                   wg1_ref, wg2_ref, bg_ref, logit_ref, sig_ref):
    x = x_ref[0]
    a = _norm_act_bf16(h_ref[0], sc_ref, sh_ref)
    e = jnp.dot(a, w5_ref[...],
                preferred_element_type=F32) + b5_ref[0]
    ri = jax.lax.broadcasted_iota(jnp.int32, (NP, 1), 0)
    le = e + jnp.where(ri == 0, -1e8, 0.0) + jnp.where(ri >= NV, -1e9, 0.0)
    mx = jnp.max(le, axis=0, keepdims=True)
    ex = jnp.exp(le - mx)
    w = ex / jnp.sum(ex, axis=0, keepdims=True)
    y = jax.lax.dot_general(w, x, (((0,), (0,)), ((), ())),
                            preferred_element_type=F32)
    gl = (jnp.dot(x[0:1, :], wg1_ref[...], preferred_element_type=F32)
          + jnp.dot(y, wg2_ref[...], preferred_element_type=F32)
          + bg_ref[0])
    logit_ref[0, 0, :] = gl[0]
    sig_ref[0, 0, :] = (1.0 / (1.0 + jnp.exp(-gl)))[0]


def _bn_act_kernel(g_ref, sc_ref, sh_ref, out_ref):
    ni = jax.lax.broadcasted_iota(jnp.int32, (NP, 1), 0)
    vmask = (ni < NV).astype(F32)
    out_ref[0] = _leaky(g_ref[0] * sc_ref[0] + sh_ref[0]) * vmask


def _bspec(shape):
    nd = len(shape)
    return pl.BlockSpec(shape, lambda b: (0,) * nd)


def _inv_correction(cinv, sc, sh, w):
    t = cinv.astype(BF16) * sc[0].astype(BF16) + sh[0].astype(BF16)
    a_inv = jnp.maximum(t, t * BF16(0.01))
    return jnp.dot(a_inv, w, preferred_element_type=F32)


def _finalize(s, q, b, g, beta, cnt):
    mean = (s + cnt * b) / cnt
    ex2 = (q + 2.0 * b * s + cnt * b * b) / cnt
    var = ex2 - mean * mean
    sc = g * jax.lax.rsqrt(var + 1e-5)
    sh = (b - mean) * sc + beta
    return sc.reshape(1, -1), sh.reshape(1, -1)


def _wcompute_gconv(xp, wc, gcp, last):
    B, _, F = xp.shape
    ws = [w.astype(BF16) for w in wc["w"]]
    bs = wc["b"]
    cnt = float(B * NV * NV)
    dims = [w.shape[1] for w in wc["w"]]

    def rspec(shape):
        return pl.BlockSpec((1,) + tuple(shape[1:]),
                            lambda b: (b,) + (0,) * (len(shape) - 1))

    ninv = float(B * (NR - NV * NV))
    h_shape = (B, NR, dims[0])
    s_shape = (B, 1, dims[0])
    h1, s, q = pl.pallas_call(
        _p1_kernel, grid=(B,), **_CP,
        in_specs=[rspec(xp.shape), _bspec(ws[0].shape)],
        out_specs=[rspec(h_shape), rspec(s_shape), rspec(s_shape)],
        out_shape=[jax.ShapeDtypeStruct(h_shape, BF16),
                   jax.ShapeDtypeStruct(s_shape, F32),
                   jax.ShapeDtypeStruct(s_shape, F32)])(xp, ws[0])
    sc, sh = _finalize(jnp.sum(s[:, 0, :], 0), jnp.sum(q[:, 0, :], 0),
                       bs[0], wc["g"][0], wc["beta"][0], cnt)
    cinv = jnp.zeros((dims[0],), F32)

    h_prev = h1
    for k in range(1, 4):
        ho_shape = (B, NR, dims[k])
        so_shape = (B, 1, dims[k])
        h_next, s, q = pl.pallas_call(
            _mid_kernel, grid=(B,), **_CP,
            in_specs=[rspec(h_prev.shape), _bspec(sc.shape),
                      _bspec(sh.shape), _bspec(ws[k].shape)],
            out_specs=[rspec(ho_shape), rspec(so_shape), rspec(so_shape)],
            out_shape=[jax.ShapeDtypeStruct(ho_shape, BF16),
                       jax.ShapeDtypeStruct(so_shape, F32),
                       jax.ShapeDtypeStruct(so_shape, F32)])(
                h_prev, sc, sh, ws[k])
        cinv = _inv_correction(cinv, sc, sh, ws[k])
        s_tot = jnp.sum(s[:, 0, :], 0) - ninv * cinv
        q_tot = jnp.sum(q[:, 0, :], 0) - ninv * cinv * cinv
        sc, sh = _finalize(s_tot, q_tot, bs[k], wc["g"][k],
                           wc["beta"][k], cnt)
        h_prev = h_next

    b5 = bs[4].reshape(1, -1)
    wg1, wg2 = gcp["w"][:F], gcp["w"][F:]
    bg = gcp["b"].reshape(1, -1)
    Fo = wg1.shape[1]

    if last:
        o_shape = (B, 1, Fo)
        logits, sig = pl.pallas_call(
            _p5last_kernel, grid=(B,), **_CP,
            in_specs=[rspec(xp.shape),
                      pl.BlockSpec((1, NP, dims[3]), lambda b: (b, 0, 0)),
                      _bspec(sc.shape), _bspec(sh.shape),
                      _bspec(ws[4].shape), _bspec(b5.shape),
                      _bspec(wg1.shape), _bspec(wg2.shape), _bspec(bg.shape)],
            out_specs=[rspec(o_shape), rspec(o_shape)],
            out_shape=[jax.ShapeDtypeStruct(o_shape, F32)] * 2)(
                xp, h_prev, sc, sh, ws[4], b5, wg1, wg2, bg)
        return logits[:, 0, :], sig[:, 0, :]

    e_shape = (B, NR, 1)
    e_col = pl.pallas_call(
        _p5a_kernel, grid=(B,), **_CP,
        in_specs=[rspec(h_prev.shape), _bspec(sc.shape), _bspec(sh.shape),
                  _bspec(ws[4].shape), _bspec(b5.shape)],
        out_specs=rspec(e_shape),
        out_shape=jax.ShapeDtypeStruct(e_shape, F32))(
            h_prev, sc, sh, ws[4], b5)
    e_grid = e_col.reshape(B, NP, NP)

    gout_shape = (B, NP, Fo)
    gs_shape = (B, 1, Fo)
    gout, gs, gq = pl.pallas_call(
        _p5b_kernel, grid=(B,), **_CP,
        in_specs=[rspec(xp.shape), rspec(e_grid.shape),
                  _bspec(wg1.shape), _bspec(wg2.shape), _bspec(bg.shape)],
        out_specs=[rspec(gout_shape), rspec(gs_shape), rspec(gs_shape)],
        out_shape=[jax.ShapeDtypeStruct(gout_shape, F32),
                   jax.ShapeDtypeStruct(gs_shape, F32),
                   jax.ShapeDtypeStruct(gs_shape, F32)])(
            xp, e_grid, wg1, wg2, bg)
    gsc, gsh = _finalize(jnp.sum(gs[:, 0, :], 0), jnp.sum(gq[:, 0, :], 0),
                         jnp.zeros((Fo,), F32), gcp["g"], gcp["beta"],
                         float(B * NV))
    act = pl.pallas_call(
        _bn_act_kernel, grid=(B,), **_CP,
        in_specs=[rspec(gout_shape), _bspec(gsc.shape), _bspec(gsh.shape)],
        out_specs=rspec(gout_shape),
        out_shape=jax.ShapeDtypeStruct(gout_shape, F32))(gout, gsc, gsh)
    return act


def kernel(z, zi_s, labels_yi, params):
    B = z.shape[0]
    zero_pad = jnp.zeros((1, B, labels_yi.shape[2]), dtype=labels_yi.dtype)
    lab_all = jnp.concatenate([zero_pad, labels_yi], axis=0)
    z_all = jnp.concatenate([z[None], zi_s], axis=0)
    nodes = jnp.transpose(jnp.concatenate([z_all, lab_all], axis=2), (1, 0, 2))
    xp = jnp.pad(nodes, ((0, 0), (0, NP - NV), (0, 0)))
    for i in range(2):
        act = _wcompute_gconv(xp, params["wc"][i], params["gc"][i], last=False)
        xp = jnp.concatenate([xp, act], axis=2)
    logits, sig = _wcompute_gconv(xp, params["wc"][2], params["gc"][2],
                                  last=True)
    return (sig, logits)
